# baseline (speedup 1.0000x reference)
import functools
import math

import jax
import jax.numpy as jnp
import numpy as np
from jax import lax
from jax.experimental import pallas as pl
from jax.experimental.pallas import tpu as pltpu
from jax.experimental.pallas import tpu_sc as plsc

_B, _P, _K = 8, 1024, 32
_SCALE = 1.0 / math.sqrt(1.0 + 1e-5)


def _fused_mlp_body(nl, relu_last, logsoftmax, h_ref, *refs):
    out_ref = refs[-1]
    a = h_ref[...]
    for i in range(nl):
        w = refs[2 * i][...]
        b = refs[2 * i + 1][...]
        a = jnp.dot(a, w, preferred_element_type=jnp.float32) + b
        if i < nl - 1 or relu_last:
            a = jnp.maximum(a * _SCALE, 0.0)
    if logsoftmax:
        m = jnp.max(a, axis=-1, keepdims=True)
        s = a - m
        lse = jnp.log(jnp.sum(jnp.exp(s), axis=-1, keepdims=True))
        a = s - lse
    out_ref[...] = a


def _mlp_pallas(params, h, blk=1024, relu_last=False, logsoftmax=False):
    m, din = h.shape
    nl = len(params)
    dout = params[-1][0].shape[1]
    assert m % blk == 0, (m, blk)
    wb = []
    in_specs = [pl.BlockSpec((blk, din), lambda i: (i, 0))]
    for w, b in params:
        wb.append(w)
        wb.append(b.reshape(1, -1))
        in_specs.append(pl.BlockSpec(w.shape, lambda i: (0, 0)))
        in_specs.append(pl.BlockSpec((1, b.shape[0]), lambda i: (0, 0)))
    return pl.pallas_call(
        functools.partial(_fused_mlp_body, nl, relu_last, logsoftmax),
        grid=(m // blk,),
        in_specs=in_specs,
        out_specs=pl.BlockSpec((blk, dout), lambda i: (i, 0)),
        out_shape=jax.ShapeDtypeStruct((m, dout), jnp.float32),
    )(h, *wb)


_T = 1024


def _radius_conv_body(r2, h3, z_ref, q_ref, idx_ref, d2_ref,
                      w2_ref, b2_ref, w3_ref, b3_ref, out_ref):
    zb = z_ref[0]
    qb = q_ref[0]
    idx = idx_ref[0]
    d2 = d2_ref[0]
    w2, b2 = w2_ref[...], b2_ref[...]
    w3, b3 = w3_ref[...], b3_ref[...]
    iota = jax.lax.broadcasted_iota(jnp.int32, (_T, _P), 1)
    z_hi = zb.astype(jnp.bfloat16)
    z_lo = (zb - z_hi.astype(jnp.float32)).astype(jnp.bfloat16)
    m = jnp.full((_T, h3), -jnp.inf, dtype=jnp.float32)
    for k in range(_K):
        oh = (iota == idx[:, k:k + 1]).astype(jnp.bfloat16)
        g = (jnp.dot(oh, z_hi, preferred_element_type=jnp.float32)
             + jnp.dot(oh, z_lo, preferred_element_type=jnp.float32))
        a = jnp.maximum((g - qb) * _SCALE, 0.0)
        a = jnp.dot(a, w2, preferred_element_type=jnp.float32) + b2
        a = jnp.maximum(a * _SCALE, 0.0)
        a = jnp.dot(a, w3, preferred_element_type=jnp.float32) + b3
        valid = d2[:, k:k + 1] <= r2
        m = jnp.maximum(m, jnp.where(valid, a, -jnp.inf))
    out_ref[0] = m


def _radius_conv_pallas(feat, pos3, idx, d2k, params, r):
    (w1, b1), (w2, b2), (w3, b3) = params
    f = feat.shape[1]
    h1 = w1.shape[1]
    h3 = w3.shape[1]
    wz = jnp.concatenate([w1, jnp.concatenate(
        [jnp.zeros((f, h1), jnp.float32), w1[f:]], axis=0)], axis=1)
    bz = jnp.concatenate([b1, jnp.zeros((h1,), jnp.float32)])
    zq = _mlp_pallas([(wz, bz)], jnp.concatenate([feat, pos3], axis=1),
                     blk=4096)
    z = zq[:, :h1].reshape(_B, _P, h1)
    q = zq[:, h1:].reshape(_B, _P, h1)
    out = pl.pallas_call(
        functools.partial(_radius_conv_body, r * r + 1e-12, h3),
        grid=(_B, _P // _T),
        in_specs=[
            pl.BlockSpec((1, _P, h1), lambda b, t: (b, 0, 0)),
            pl.BlockSpec((1, _T, h1), lambda b, t: (b, t, 0)),
            pl.BlockSpec((1, _T, _K), lambda b, t: (b, t, 0)),
            pl.BlockSpec((1, _T, _K), lambda b, t: (b, t, 0)),
            pl.BlockSpec(w2.shape, lambda b, t: (0, 0)),
            pl.BlockSpec((1, w2.shape[1]), lambda b, t: (0, 0)),
            pl.BlockSpec(w3.shape, lambda b, t: (0, 0)),
            pl.BlockSpec((1, h3), lambda b, t: (0, 0)),
        ],
        out_specs=pl.BlockSpec((1, _T, h3), lambda b, t: (b, t, 0)),
        out_shape=jax.ShapeDtypeStruct((_B, _P, h3), jnp.float32),
    )(z, q, idx, d2k, w2, b2.reshape(1, -1), w3, b3.reshape(1, -1))
    return out.reshape(_B * _P, h3)


_TN = 1024


def _knn_body(pt_ref, pa_ref, idx_ref, d2_ref):
    pt = pt_ref[0]
    pa = pa_ref[0]
    xt, yt = pt[:, 0:1], pt[:, 1:2]
    xa, ya = pa[0:1, :], pa[1:2, :]
    dx = xt - xa
    dy = yt - ya
    d2 = dx * dx + dy * dy
    iota = jax.lax.broadcasted_iota(jnp.int32, (_TN, _P), 1)
    idxs, d2s = [], []
    for k in range(_K):
        mn = jnp.min(d2, axis=1, keepdims=True)
        sel = jnp.where(d2 == mn, iota, _P)
        amin = jnp.min(sel, axis=1, keepdims=True)
        idxs.append(amin)
        d2s.append(mn)
        if k < _K - 1:
            d2 = jnp.where(iota == amin, jnp.inf, d2)
    idx_ref[0] = jnp.concatenate(idxs, axis=1)
    d2_ref[0] = jnp.concatenate(d2s, axis=1)


def _neighbors(pos):
    pb = pos.reshape(_B, _P, 2)
    pt = jnp.transpose(pb, (0, 2, 1))
    return pl.pallas_call(
        _knn_body,
        grid=(_B, _P // _TN),
        in_specs=[
            pl.BlockSpec((1, _TN, 2), lambda b, t: (b, t, 0)),
            pl.BlockSpec((1, 2, _P), lambda b, t: (b, 0, 0)),
        ],
        out_specs=[
            pl.BlockSpec((1, _TN, _K), lambda b, t: (b, t, 0)),
            pl.BlockSpec((1, _TN, _K), lambda b, t: (b, t, 0)),
        ],
        out_shape=[
            jax.ShapeDtypeStruct((_B, _P, _K), jnp.int32),
            jax.ShapeDtypeStruct((_B, _P, _K), jnp.float32),
        ],
    )(pb, pt)


def _sc_num_workers():
    try:
        info = plsc.get_sparse_core_info()
        return info.num_cores, info.num_subcores
    except Exception:
        return 2, 16


_SC_CH = 128


def _sc_gather_rows(table, fidx, d):
    n = fidx.shape[0]
    nc, ns = _sc_num_workers()
    nw = nc * ns
    per_w = n // nw
    assert n % nw == 0 and per_w % _SC_CH == 0, (n, nw)
    mesh = plsc.VectorSubcoreMesh(core_axis_name="c", subcore_axis_name="s")

    @functools.partial(
        pl.kernel, mesh=mesh,
        out_type=jax.ShapeDtypeStruct((n, d), jnp.float32),
        scratch_types=[
            pltpu.VMEM((_SC_CH,), jnp.int32),
            pltpu.VMEM((_SC_CH, d), jnp.float32),
            pltpu.SemaphoreType.DMA,
        ],
    )
    def gk(table_hbm, idx_hbm, out_hbm, idx_v, rows_v, sem):
        wid = lax.axis_index("s") * nc + lax.axis_index("c")
        base = wid * per_w
        for c in range(per_w // _SC_CH):
            off = base + c * _SC_CH
            pltpu.sync_copy(idx_hbm.at[pl.ds(off, _SC_CH)], idx_v)
            pltpu.async_copy(table_hbm.at[idx_v], rows_v, sem).wait()
            pltpu.sync_copy(rows_v, out_hbm.at[pl.ds(off, _SC_CH)])

    return gk(table, fidx)


def _interp_body(r0_ref, r1_ref, r2_ref, d2_ref, out_ref):
    w = 1.0 / jnp.maximum(d2_ref[...], 1e-16)
    s = jnp.sum(w, axis=1, keepdims=True)
    out_ref[...] = (r0_ref[...] * w[:, 0:1] + r1_ref[...] * w[:, 1:2]
                    + r2_ref[...] * w[:, 2:3]) / s


def _interp_pallas(rows, d23, d):
    n = d23.shape[0]
    blk = 1024
    nb = n // blk
    return pl.pallas_call(
        _interp_body,
        grid=(nb,),
        in_specs=[
            pl.BlockSpec((blk, d), lambda i: (i, 0)),
            pl.BlockSpec((blk, d), lambda i: (i + nb, 0)),
            pl.BlockSpec((blk, d), lambda i: (i + 2 * nb, 0)),
            pl.BlockSpec((blk, 3), lambda i: (i, 0)),
        ],
        out_specs=pl.BlockSpec((blk, d), lambda i: (i, 0)),
        out_shape=jax.ShapeDtypeStruct((n, d), jnp.float32),
    )(rows, rows, rows, d23)


def kernel(x, pos, batch, sa1_params, sa2_params, sa3_params,
           fp3_params, fp2_params, fp1_params, head_params):
    del batch
    idx, d2k = _neighbors(pos)
    bidx = jnp.arange(_B)[:, None, None]
    fidx3 = jnp.transpose(idx[:, :, :3] + bidx * _P, (2, 0, 1)).reshape(-1)
    d23 = d2k[:, :, :3].reshape(_B * _P, 3)

    x1 = _radius_conv_pallas(x, pos, idx, d2k, sa1_params, 0.1)
    x2 = _radius_conv_pallas(x1, pos, idx, d2k, sa2_params, 0.2)

    g = _mlp_pallas(sa3_params, jnp.concatenate([x2, pos], axis=1), blk=1024)
    x3 = jnp.max(g.reshape(_B, _P, -1), axis=1)

    up3 = jnp.broadcast_to(x3[:, None, :], (_B, _P, x3.shape[-1])).reshape(_B * _P, -1)
    h3 = _mlp_pallas(fp3_params, jnp.concatenate([up3, x2], axis=1), blk=1024)

    def interp(feat):
        d = feat.shape[1]
        rows = _sc_gather_rows(feat, fidx3, d)
        return _interp_pallas(rows, d23, d)

    up2 = interp(h3)
    h2 = _mlp_pallas(fp2_params, jnp.concatenate([up2, x1], axis=1), blk=1024)
    up1 = interp(h2)
    h1 = _mlp_pallas(fp1_params, jnp.concatenate([up1, x], axis=1), blk=1024)
    return _mlp_pallas(head_params, h1, blk=1024, logsoftmax=True)


# speedup vs baseline: 1.0493x; 1.0493x over previous
import functools
import math

import jax
import jax.numpy as jnp
import numpy as np
from jax import lax
from jax.experimental import pallas as pl
from jax.experimental.pallas import tpu as pltpu
from jax.experimental.pallas import tpu_sc as plsc

_B, _P, _K = 8, 1024, 32
_SCALE = 1.0 / math.sqrt(1.0 + 1e-5)


def _fused_mlp_body(nl, relu_last, logsoftmax, h_ref, *refs):
    out_ref = refs[-1]
    a = h_ref[...]
    for i in range(nl):
        w = refs[2 * i][...]
        b = refs[2 * i + 1][...]
        a = jnp.dot(a, w, preferred_element_type=jnp.float32) + b
        if i < nl - 1 or relu_last:
            a = jnp.maximum(a * _SCALE, 0.0)
    if logsoftmax:
        m = jnp.max(a, axis=-1, keepdims=True)
        s = a - m
        lse = jnp.log(jnp.sum(jnp.exp(s), axis=-1, keepdims=True))
        a = s - lse
    out_ref[...] = a


def _mlp_pallas(params, h, blk=1024, relu_last=False, logsoftmax=False):
    m, din = h.shape
    nl = len(params)
    dout = params[-1][0].shape[1]
    assert m % blk == 0, (m, blk)
    wb = []
    in_specs = [pl.BlockSpec((blk, din), lambda i: (i, 0))]
    for w, b in params:
        wb.append(w)
        wb.append(b.reshape(1, -1))
        in_specs.append(pl.BlockSpec(w.shape, lambda i: (0, 0)))
        in_specs.append(pl.BlockSpec((1, b.shape[0]), lambda i: (0, 0)))
    return pl.pallas_call(
        functools.partial(_fused_mlp_body, nl, relu_last, logsoftmax),
        grid=(m // blk,),
        in_specs=in_specs,
        out_specs=pl.BlockSpec((blk, dout), lambda i: (i, 0)),
        out_shape=jax.ShapeDtypeStruct((m, dout), jnp.float32),
    )(h, *wb)


_T = 1024


def _radius_conv_body(r2, h3, z_ref, q_ref, idx_ref, d2_ref,
                      w2_ref, b2_ref, w3_ref, b3_ref, out_ref):
    zb = z_ref[0]
    qb = q_ref[0]
    idx = idx_ref[0]
    d2 = d2_ref[0]
    w2, b2 = w2_ref[...], b2_ref[...]
    w3, b3 = w3_ref[...], b3_ref[...]
    iota = jax.lax.broadcasted_iota(jnp.int32, (_T, _P), 1)
    z_hi = zb.astype(jnp.bfloat16)
    z_lo = (zb - z_hi.astype(jnp.float32)).astype(jnp.bfloat16)
    m = jnp.full((_T, h3), -jnp.inf, dtype=jnp.float32)
    for k in range(_K):
        oh = (iota == idx[:, k:k + 1]).astype(jnp.bfloat16)
        g = (jnp.dot(oh, z_hi, preferred_element_type=jnp.float32)
             + jnp.dot(oh, z_lo, preferred_element_type=jnp.float32))
        a = jnp.maximum((g - qb) * _SCALE, 0.0)
        a = jnp.dot(a, w2, preferred_element_type=jnp.float32) + b2
        a = jnp.maximum(a * _SCALE, 0.0)
        a = jnp.dot(a, w3, preferred_element_type=jnp.float32) + b3
        valid = d2[:, k:k + 1] <= r2
        m = jnp.maximum(m, jnp.where(valid, a, -jnp.inf))
    out_ref[0] = m


def _radius_conv_pallas(feat, pos3, idx, d2k, params, r):
    (w1, b1), (w2, b2), (w3, b3) = params
    f = feat.shape[1]
    h1 = w1.shape[1]
    h3 = w3.shape[1]
    wz = jnp.concatenate([w1, jnp.concatenate(
        [jnp.zeros((f, h1), jnp.float32), w1[f:]], axis=0)], axis=1)
    bz = jnp.concatenate([b1, jnp.zeros((h1,), jnp.float32)])
    zq = _mlp_pallas([(wz, bz)], jnp.concatenate([feat, pos3], axis=1),
                     blk=4096)
    z = zq[:, :h1].reshape(_B, _P, h1)
    q = zq[:, h1:].reshape(_B, _P, h1)
    out = pl.pallas_call(
        functools.partial(_radius_conv_body, r * r + 1e-12, h3),
        grid=(_B, _P // _T),
        in_specs=[
            pl.BlockSpec((1, _P, h1), lambda b, t: (b, 0, 0)),
            pl.BlockSpec((1, _T, h1), lambda b, t: (b, t, 0)),
            pl.BlockSpec((1, _T, _K), lambda b, t: (b, t, 0)),
            pl.BlockSpec((1, _T, _K), lambda b, t: (b, t, 0)),
            pl.BlockSpec(w2.shape, lambda b, t: (0, 0)),
            pl.BlockSpec((1, w2.shape[1]), lambda b, t: (0, 0)),
            pl.BlockSpec(w3.shape, lambda b, t: (0, 0)),
            pl.BlockSpec((1, h3), lambda b, t: (0, 0)),
        ],
        out_specs=pl.BlockSpec((1, _T, h3), lambda b, t: (b, t, 0)),
        out_shape=jax.ShapeDtypeStruct((_B, _P, h3), jnp.float32),
    )(z, q, idx, d2k, w2, b2.reshape(1, -1), w3, b3.reshape(1, -1))
    return out.reshape(_B * _P, h3)


_TN = 512


def _knn_body(pt_ref, pa_ref, idx_ref, d2_ref):
    pt = pt_ref[0]
    pa = pa_ref[0]
    xt, yt = pt[:, 0:1], pt[:, 1:2]
    xa, ya = pa[0:1, :], pa[1:2, :]
    dx = xt - xa
    dy = yt - ya
    d2 = dx * dx + dy * dy
    iota = jax.lax.broadcasted_iota(jnp.int32, (_TN, _P), 1)
    idxs, d2s = [], []
    for k in range(_K):
        mn = jnp.min(d2, axis=1, keepdims=True)
        sel = jnp.where(d2 == mn, iota, _P)
        amin = jnp.min(sel, axis=1, keepdims=True)
        idxs.append(amin)
        d2s.append(mn)
        if k < _K - 1:
            d2 = jnp.where(iota == amin, jnp.inf, d2)
    idx_ref[0] = jnp.concatenate(idxs, axis=1)
    d2_ref[0] = jnp.concatenate(d2s, axis=1)


def _neighbors(pos):
    pb = pos.reshape(_B, _P, 2)
    pt = jnp.transpose(pb, (0, 2, 1))
    return pl.pallas_call(
        _knn_body,
        grid=(_B, _P // _TN),
        in_specs=[
            pl.BlockSpec((1, _TN, 2), lambda b, t: (b, t, 0)),
            pl.BlockSpec((1, 2, _P), lambda b, t: (b, 0, 0)),
        ],
        out_specs=[
            pl.BlockSpec((1, _TN, _K), lambda b, t: (b, t, 0)),
            pl.BlockSpec((1, _TN, _K), lambda b, t: (b, t, 0)),
        ],
        out_shape=[
            jax.ShapeDtypeStruct((_B, _P, _K), jnp.int32),
            jax.ShapeDtypeStruct((_B, _P, _K), jnp.float32),
        ],
    )(pb, pt)


def _sc_num_workers():
    try:
        info = plsc.get_sparse_core_info()
        return info.num_cores, info.num_subcores
    except Exception:
        return 2, 16


_SC_CH = 128


def _sc_gather_rows(table, fidx, d):
    n = fidx.shape[0]
    nc, ns = _sc_num_workers()
    nw = nc * ns
    per_w = n // nw
    assert n % nw == 0 and per_w % _SC_CH == 0, (n, nw)
    mesh = plsc.VectorSubcoreMesh(core_axis_name="c", subcore_axis_name="s")

    @functools.partial(
        pl.kernel, mesh=mesh,
        out_type=jax.ShapeDtypeStruct((n, d), jnp.float32),
        scratch_types=[
            pltpu.VMEM((_SC_CH,), jnp.int32),
            pltpu.VMEM((_SC_CH, d), jnp.float32),
            pltpu.SemaphoreType.DMA,
        ],
    )
    def gk(table_hbm, idx_hbm, out_hbm, idx_v, rows_v, sem):
        wid = lax.axis_index("s") * nc + lax.axis_index("c")
        base = wid * per_w
        for c in range(per_w // _SC_CH):
            off = base + c * _SC_CH
            pltpu.sync_copy(idx_hbm.at[pl.ds(off, _SC_CH)], idx_v)
            pltpu.async_copy(table_hbm.at[idx_v], rows_v, sem).wait()
            pltpu.sync_copy(rows_v, out_hbm.at[pl.ds(off, _SC_CH)])

    return gk(table, fidx)


def _interp_body(r0_ref, r1_ref, r2_ref, d2_ref, out_ref):
    w = 1.0 / jnp.maximum(d2_ref[...], 1e-16)
    s = jnp.sum(w, axis=1, keepdims=True)
    out_ref[...] = (r0_ref[...] * w[:, 0:1] + r1_ref[...] * w[:, 1:2]
                    + r2_ref[...] * w[:, 2:3]) / s


def _interp_pallas(rows, d23, d):
    n = d23.shape[0]
    blk = 1024
    nb = n // blk
    return pl.pallas_call(
        _interp_body,
        grid=(nb,),
        in_specs=[
            pl.BlockSpec((blk, d), lambda i: (i, 0)),
            pl.BlockSpec((blk, d), lambda i: (i + nb, 0)),
            pl.BlockSpec((blk, d), lambda i: (i + 2 * nb, 0)),
            pl.BlockSpec((blk, 3), lambda i: (i, 0)),
        ],
        out_specs=pl.BlockSpec((blk, d), lambda i: (i, 0)),
        out_shape=jax.ShapeDtypeStruct((n, d), jnp.float32),
    )(rows, rows, rows, d23)


def kernel(x, pos, batch, sa1_params, sa2_params, sa3_params,
           fp3_params, fp2_params, fp1_params, head_params):
    del batch
    idx, d2k = _neighbors(pos)
    bidx = jnp.arange(_B)[:, None, None]
    fidx3 = jnp.transpose(idx[:, :, :3] + bidx * _P, (2, 0, 1)).reshape(-1)
    d23 = d2k[:, :, :3].reshape(_B * _P, 3)

    x1 = _radius_conv_pallas(x, pos, idx, d2k, sa1_params, 0.1)
    x2 = _radius_conv_pallas(x1, pos, idx, d2k, sa2_params, 0.2)

    g = _mlp_pallas(sa3_params, jnp.concatenate([x2, pos], axis=1), blk=1024)
    x3 = jnp.max(g.reshape(_B, _P, -1), axis=1)

    up3 = jnp.broadcast_to(x3[:, None, :], (_B, _P, x3.shape[-1])).reshape(_B * _P, -1)
    h3 = _mlp_pallas(fp3_params, jnp.concatenate([up3, x2], axis=1), blk=1024)

    def interp(feat):
        d = feat.shape[1]
        rows = _sc_gather_rows(feat, fidx3, d)
        return _interp_pallas(rows, d23, d)

    up2 = interp(h3)
    h2 = _mlp_pallas(fp2_params, jnp.concatenate([up2, x1], axis=1), blk=1024)
    up1 = interp(h2)
    h1 = _mlp_pallas(fp1_params, jnp.concatenate([up1, x], axis=1), blk=1024)
    return _mlp_pallas(head_params, h1, blk=1024, logsoftmax=True)
